# Initial kernel scaffold; baseline (speedup 1.0000x reference)
import jax
import jax.numpy as jnp
from jax.experimental import pallas as pl


def kernel(condition, table):
    raise NotImplementedError("write your pallas kernel here")


# speedup vs baseline: 5.0253x; 5.0253x over previous
import functools

import jax
import jax.numpy as jnp
from jax import lax
from jax.experimental import pallas as pl
from jax.experimental.pallas import tpu as pltpu
from jax.experimental.pallas import tpu_sc as plsc

NC, NS = 2, 16
NW = NC * NS
BATCH, HIST, EMB = 16384, 200, 32
B = BATCH * HIST
BPW = B // NW
CHUNK = 2048
NCHUNK = BPW // CHUNK

_mesh = plsc.VectorSubcoreMesh(core_axis_name="c", subcore_axis_name="s")


@functools.partial(
    pl.kernel,
    out_type=jax.ShapeDtypeStruct((B, EMB), jnp.float32),
    mesh=_mesh,
    scratch_types=[
        pltpu.VMEM((CHUNK,), jnp.int32),
        pltpu.VMEM((CHUNK, EMB), jnp.float32),
        pltpu.SemaphoreType.DMA,
    ],
    compiler_params=pltpu.CompilerParams(use_tc_tiling_on_sc=False),
)
def _gather_kernel(idx_hbm, table_hbm, out_hbm, idx_v, rows_v, sem):
    wid = lax.axis_index("s") * NC + lax.axis_index("c")
    base = wid * BPW

    def body(i, _):
        off = base + i * CHUNK
        pltpu.sync_copy(idx_hbm.at[pl.ds(off, CHUNK)], idx_v)
        pltpu.async_copy(table_hbm.at[idx_v], rows_v, sem).wait()
        pltpu.sync_copy(rows_v, out_hbm.at[pl.ds(off, CHUNK)])
        return ()

    lax.fori_loop(0, NCHUNK, body, ())


@jax.jit
def kernel(condition, table):
    idx = condition.reshape(-1).astype(jnp.int32)
    out = _gather_kernel(idx, table)
    return out.reshape(BATCH, HIST, EMB)


# speedup vs baseline: 5.1178x; 1.0184x over previous
import functools

import jax
import jax.numpy as jnp
from jax import lax
from jax.experimental import pallas as pl
from jax.experimental.pallas import tpu as pltpu
from jax.experimental.pallas import tpu_sc as plsc

NC, NS = 2, 16
NW = NC * NS
BATCH, HIST, EMB = 16384, 200, 32
B = BATCH * HIST
BPW = B // NW
CHUNK = 1600
NCHUNK = BPW // CHUNK
NPAIR = NCHUNK // 2

_mesh = plsc.VectorSubcoreMesh(core_axis_name="c", subcore_axis_name="s")


@functools.partial(
    pl.kernel,
    out_type=jax.ShapeDtypeStruct((B, EMB), jnp.float32),
    mesh=_mesh,
    scratch_types=[
        pltpu.VMEM((2, CHUNK), jnp.int32),
        pltpu.VMEM((2, CHUNK, EMB), jnp.float32),
        pltpu.SemaphoreType.DMA((2,)),
        pltpu.SemaphoreType.DMA,
        pltpu.SemaphoreType.DMA((2,)),
    ],
    compiler_params=pltpu.CompilerParams(use_tc_tiling_on_sc=False),
)
def _gather_kernel(idx_hbm, table_hbm, out_hbm, idx_v, rows_v, semi, semg, semo):
    wid = lax.axis_index("s") * NC + lax.axis_index("c")
    base = wid * BPW

    for b in range(2):
        pltpu.async_copy(
            idx_hbm.at[pl.ds(base + b * CHUNK, CHUNK)], idx_v.at[b], semi.at[b]
        )

    def pair(t, _):
        for b in range(2):
            off = base + (2 * t + b) * CHUNK

            @pl.when(t > 0)
            def _():
                pltpu.make_async_copy(
                    rows_v.at[b],
                    out_hbm.at[pl.ds(off - 2 * CHUNK, CHUNK)],
                    semo.at[b],
                ).wait()

            pltpu.make_async_copy(
                idx_hbm.at[pl.ds(off, CHUNK)], idx_v.at[b], semi.at[b]
            ).wait()

            pltpu.async_copy(table_hbm.at[idx_v.at[b]], rows_v.at[b], semg).wait()

            @pl.when(t < NPAIR - 1)
            def _():
                pltpu.async_copy(
                    idx_hbm.at[pl.ds(off + 2 * CHUNK, CHUNK)],
                    idx_v.at[b],
                    semi.at[b],
                )

            pltpu.async_copy(rows_v.at[b], out_hbm.at[pl.ds(off, CHUNK)], semo.at[b])
        return ()

    lax.fori_loop(0, NPAIR, pair, ())

    for b in range(2):
        off = base + (2 * (NPAIR - 1) + b) * CHUNK
        pltpu.make_async_copy(
            rows_v.at[b], out_hbm.at[pl.ds(off, CHUNK)], semo.at[b]
        ).wait()


@jax.jit
def kernel(condition, table):
    idx = condition.reshape(-1).astype(jnp.int32)
    out = _gather_kernel(idx, table)
    return out.reshape(BATCH, HIST, EMB)
